# baseline (speedup 1.0000x reference)
import functools
import jax
import jax.numpy as jnp
from jax import lax
from jax.experimental import pallas as pl
from jax.experimental.pallas import tpu as pltpu
from jax.experimental.pallas import tpu_sc as plsc

B, N, D = 8, 1024, 256
K = 8192
Q = 4
COMMIT_W = 0.02
BN = B * N
BN_TILE = 256
NBN = BN // BN_TILE
KC = 8192
NKC = K // KC


def _score_kernel(r_ref, qprev_ref, cbT_ref, rout_ref, idx_ref, loss_ref):
    r = r_ref[...] - qprev_ref[...]
    rout_ref[...] = r
    loss_ref[0, :, :] = jnp.sum(r * r, axis=1, keepdims=True)

    rb = r.astype(jnp.bfloat16)
    run_m = jnp.full((BN_TILE, 1), jnp.inf, jnp.float32)
    run_i = jnp.zeros((BN_TILE, 1), jnp.int32)
    for c in range(NKC):
        cbT_c = cbT_ref[0, :, c * KC:(c + 1) * KC]
        cn = jnp.sum(cbT_c * cbT_c, axis=0, keepdims=True)
        s = cn - 2.0 * jnp.dot(rb, cbT_c.astype(jnp.bfloat16),
                               preferred_element_type=jnp.float32)
        m = jnp.min(s, axis=1, keepdims=True)
        cols = jax.lax.broadcasted_iota(jnp.int32, (BN_TILE, KC), 1) + c * KC
        a = jnp.min(jnp.where(s == m, cols, K), axis=1, keepdims=True)
        upd = m < run_m
        run_i = jnp.where(upd, a, run_i)
        run_m = jnp.where(upd, m, run_m)
    idx_ref[0, :, :] = run_i


def _final_kernel(x_ref, r_ref, qprev_ref, qout_ref, loss_ref):
    r = r_ref[...] - qprev_ref[...]
    loss_ref[0, :, :] = jnp.sum(r * r, axis=1, keepdims=True)
    qout_ref[...] = x_ref[...] - r


def _score_layer(r, qprev, cbT_q):
    return pl.pallas_call(
        _score_kernel,
        grid=(NBN,),
        in_specs=[
            pl.BlockSpec((BN_TILE, D), lambda bn: (bn, 0)),
            pl.BlockSpec((BN_TILE, D), lambda bn: (bn, 0)),
            pl.BlockSpec((1, D, K), lambda bn: (0, 0, 0)),
        ],
        out_specs=[
            pl.BlockSpec((BN_TILE, D), lambda bn: (bn, 0)),
            pl.BlockSpec((1, BN_TILE, 1), lambda bn: (bn, 0, 0)),
            pl.BlockSpec((1, BN_TILE, 1), lambda bn: (bn, 0, 0)),
        ],
        out_shape=[
            jax.ShapeDtypeStruct((BN, D), jnp.float32),
            jax.ShapeDtypeStruct((NBN, BN_TILE, 1), jnp.int32),
            jax.ShapeDtypeStruct((NBN, BN_TILE, 1), jnp.float32),
        ],
    )(r, qprev, cbT_q)


def _final_layer(x, r, qprev):
    return pl.pallas_call(
        _final_kernel,
        grid=(NBN,),
        in_specs=[
            pl.BlockSpec((BN_TILE, D), lambda bn: (bn, 0)),
            pl.BlockSpec((BN_TILE, D), lambda bn: (bn, 0)),
            pl.BlockSpec((BN_TILE, D), lambda bn: (bn, 0)),
        ],
        out_specs=[
            pl.BlockSpec((BN_TILE, D), lambda bn: (bn, 0)),
            pl.BlockSpec((1, BN_TILE, 1), lambda bn: (bn, 0, 0)),
        ],
        out_shape=[
            jax.ShapeDtypeStruct((BN, D), jnp.float32),
            jax.ShapeDtypeStruct((NBN, BN_TILE, 1), jnp.float32),
        ],
    )(x, r, qprev)


def _make_sc_gather():
    info = plsc.get_sparse_core_info()
    nc, ns = info.num_cores, info.num_subcores
    nw = nc * ns
    b_per_w = BN // nw
    mesh = plsc.VectorSubcoreMesh(core_axis_name="c", subcore_axis_name="s")

    @functools.partial(
        pl.kernel, mesh=mesh,
        out_type=jax.ShapeDtypeStruct((BN, D), jnp.float32),
        scratch_types=[
            pltpu.VMEM((b_per_w,), jnp.int32),
            pltpu.VMEM((b_per_w, D), jnp.float32),
            pltpu.SemaphoreType.DMA,
        ],
    )
    def sc_gather(table_hbm, idx_hbm, out_hbm, idx_v, rows_v, sem):
        wid = lax.axis_index("s") * nc + lax.axis_index("c")
        base = wid * b_per_w
        pltpu.sync_copy(idx_hbm.at[pl.ds(base, b_per_w)], idx_v)
        pltpu.async_copy(table_hbm.at[idx_v], rows_v, sem).wait()
        pltpu.sync_copy(rows_v, out_hbm.at[pl.ds(base, b_per_w)])

    return sc_gather


_sc_gather = _make_sc_gather()


def kernel(x, codebooks):
    xf = x.reshape(BN, D)
    cbT = jnp.swapaxes(codebooks, 1, 2)
    r = xf
    qprev = jnp.zeros_like(xf)
    idxs = []
    losses = []
    for q in range(Q):
        r, idx_q, lossp = _score_layer(r, qprev, cbT[q:q + 1])
        if q > 0:
            losses.append(lossp)
        idx_flat = idx_q.reshape(BN)
        qprev = _sc_gather(codebooks[q], idx_flat)
        idxs.append(idx_flat)
    qout, lossp = _final_layer(xf, r, qprev)
    losses.append(lossp)

    quantized_out = qout.reshape(B, N, D)
    indices = jnp.stack(idxs, axis=-1).reshape(B, N, Q)
    all_losses = jnp.stack([l.sum() for l in losses]) / (BN * D) * COMMIT_W
    return quantized_out, indices, all_losses, jnp.sum(all_losses)


# speedup vs baseline: 1.2367x; 1.2367x over previous
import functools
import jax
import jax.numpy as jnp
from jax import lax
from jax.experimental import pallas as pl
from jax.experimental.pallas import tpu as pltpu
from jax.experimental.pallas import tpu_sc as plsc

B, N, D = 8, 1024, 256
K = 8192
Q = 4
COMMIT_W = 0.02
BN = B * N
BN_TILE = 512
NBN = BN // BN_TILE
KC = 8192
NKC = K // KC


def _score_kernel(r_ref, qprev_ref, cbT_ref, rout_ref, idx_ref, loss_ref):
    r = r_ref[...] - qprev_ref[...]
    rout_ref[...] = r
    loss_ref[0, :, :] = jnp.sum(r * r, axis=1, keepdims=True)

    rb = r.astype(jnp.bfloat16)
    run_m = jnp.full((BN_TILE, 1), jnp.inf, jnp.float32)
    run_i = jnp.zeros((BN_TILE, 1), jnp.int32)
    for c in range(NKC):
        cbT_c = cbT_ref[0, :, c * KC:(c + 1) * KC]
        cn = jnp.sum(cbT_c * cbT_c, axis=0, keepdims=True)
        s = cn - 2.0 * jnp.dot(rb, cbT_c.astype(jnp.bfloat16),
                               preferred_element_type=jnp.float32)
        m = jnp.min(s, axis=1, keepdims=True)
        cols = jax.lax.broadcasted_iota(jnp.int32, (BN_TILE, KC), 1) + c * KC
        a = jnp.min(jnp.where(s == m, cols, K), axis=1, keepdims=True)
        upd = m < run_m
        run_i = jnp.where(upd, a, run_i)
        run_m = jnp.where(upd, m, run_m)
    idx_ref[0, :, :] = run_i


def _final_kernel(x_ref, r_ref, qprev_ref, qout_ref, loss_ref):
    r = r_ref[...] - qprev_ref[...]
    loss_ref[0, :, :] = jnp.sum(r * r, axis=1, keepdims=True)
    qout_ref[...] = x_ref[...] - r


def _score_layer(r, qprev, cbT_q):
    return pl.pallas_call(
        _score_kernel,
        grid=(NBN,),
        in_specs=[
            pl.BlockSpec((BN_TILE, D), lambda bn: (bn, 0)),
            pl.BlockSpec((BN_TILE, D), lambda bn: (bn, 0)),
            pl.BlockSpec((1, D, K), lambda bn: (0, 0, 0)),
        ],
        out_specs=[
            pl.BlockSpec((BN_TILE, D), lambda bn: (bn, 0)),
            pl.BlockSpec((1, BN_TILE, 1), lambda bn: (bn, 0, 0)),
            pl.BlockSpec((1, BN_TILE, 1), lambda bn: (bn, 0, 0)),
        ],
        out_shape=[
            jax.ShapeDtypeStruct((BN, D), jnp.float32),
            jax.ShapeDtypeStruct((NBN, BN_TILE, 1), jnp.int32),
            jax.ShapeDtypeStruct((NBN, BN_TILE, 1), jnp.float32),
        ],
    )(r, qprev, cbT_q)


def _final_layer(x, r, qprev):
    return pl.pallas_call(
        _final_kernel,
        grid=(NBN,),
        in_specs=[
            pl.BlockSpec((BN_TILE, D), lambda bn: (bn, 0)),
            pl.BlockSpec((BN_TILE, D), lambda bn: (bn, 0)),
            pl.BlockSpec((BN_TILE, D), lambda bn: (bn, 0)),
        ],
        out_specs=[
            pl.BlockSpec((BN_TILE, D), lambda bn: (bn, 0)),
            pl.BlockSpec((1, BN_TILE, 1), lambda bn: (bn, 0, 0)),
        ],
        out_shape=[
            jax.ShapeDtypeStruct((BN, D), jnp.float32),
            jax.ShapeDtypeStruct((NBN, BN_TILE, 1), jnp.float32),
        ],
    )(x, r, qprev)


def _make_sc_gather():
    info = plsc.get_sparse_core_info()
    nc, ns = info.num_cores, info.num_subcores
    nw = nc * ns
    b_per_w = BN // nw
    mesh = plsc.VectorSubcoreMesh(core_axis_name="c", subcore_axis_name="s")

    @functools.partial(
        pl.kernel, mesh=mesh,
        out_type=jax.ShapeDtypeStruct((BN, D), jnp.float32),
        scratch_types=[
            pltpu.VMEM((b_per_w,), jnp.int32),
            pltpu.VMEM((b_per_w, D), jnp.float32),
            pltpu.SemaphoreType.DMA,
        ],
    )
    def sc_gather(table_hbm, idx_hbm, out_hbm, idx_v, rows_v, sem):
        wid = lax.axis_index("s") * nc + lax.axis_index("c")
        base = wid * b_per_w
        pltpu.sync_copy(idx_hbm.at[pl.ds(base, b_per_w)], idx_v)
        pltpu.async_copy(table_hbm.at[idx_v], rows_v, sem).wait()
        pltpu.sync_copy(rows_v, out_hbm.at[pl.ds(base, b_per_w)])

    return sc_gather


_sc_gather = _make_sc_gather()


def kernel(x, codebooks):
    xf = x.reshape(BN, D)
    cbT = jnp.swapaxes(codebooks, 1, 2)
    r = xf
    qprev = jnp.zeros_like(xf)
    idxs = []
    losses = []
    for q in range(Q):
        r, idx_q, lossp = _score_layer(r, qprev, cbT[q:q + 1])
        if q > 0:
            losses.append(lossp)
        idx_flat = idx_q.reshape(BN)
        qprev = _sc_gather(codebooks[q], idx_flat)
        idxs.append(idx_flat)
    qout, lossp = _final_layer(xf, r, qprev)
    losses.append(lossp)

    quantized_out = qout.reshape(B, N, D)
    indices = jnp.stack(idxs, axis=-1).reshape(B, N, Q)
    all_losses = jnp.stack([l.sum() for l in losses]) / (BN * D) * COMMIT_W
    return quantized_out, indices, all_losses, jnp.sum(all_losses)
